# Initial kernel scaffold; baseline (speedup 1.0000x reference)
import jax
import jax.numpy as jnp
from jax.experimental import pallas as pl


def kernel(x, edge_index, batch, params):
    raise NotImplementedError("write your pallas kernel here")


# speedup vs baseline: 7.5468x; 7.5468x over previous
import functools

import jax
import jax.numpy as jnp
from jax import lax
from jax.experimental import pallas as pl
from jax.experimental.pallas import tpu as pltpu
from jax.experimental.pallas import tpu_sc as plsc

N = 10000
E = 320000
D = 128
H = 128
OUT = 64
G = 128
L = 4
EPS = 1e-5

NC = 2
NS = 16
NW = NC * NS
EW = E // NW
CSZ = 100
CHUNKS = EW // CSZ
NPAD = 10240
RPT = NPAD // NS
ZROWS = 64
FH = H // 2


def _agg_body(h0_hbm, h1_hbm, src_hbm, dst_hbm, out_hbm,
              idx_s, idx_d, rows0, rows1, zbuf, acc, sem0, sem1):
    c = lax.axis_index("c")
    s = lax.axis_index("s")
    wid = s * NC + c

    pltpu.sync_copy(src_hbm.at[wid], idx_s)
    pltpu.sync_copy(dst_hbm.at[wid], idx_d)

    def _zb(k, carry):
        zbuf[k // 4, pl.ds((k % 4) * 16, 16)] = jnp.zeros((16,), jnp.float32)
        return carry
    lax.fori_loop(0, ZROWS * 4, _zb, None)

    for ph, h_hbm in ((0, h0_hbm), (1, h1_hbm)):
        def _zc(t, carry):
            pltpu.sync_copy(zbuf, acc.at[pl.ds(s * RPT + t * ZROWS, ZROWS)])
            return carry
        lax.fori_loop(0, RPT // ZROWS, _zc, None)
        plsc.subcore_barrier()

        pltpu.async_copy(h_hbm.at[idx_s.at[0]], rows0, sem0)

        def _body(jj, carry):
            j0 = 2 * jj
            j1 = j0 + 1
            j2 = lax.rem(j0 + 2, CHUNKS)
            cp1 = pltpu.async_copy(h_hbm.at[idx_s.at[j1]], rows1, sem1)
            pltpu.make_async_copy(h_hbm.at[idx_s.at[j0]], rows0, sem0).wait()
            pltpu.sync_copy(rows0, acc.at[idx_d.at[j0]], add=True)
            pltpu.async_copy(h_hbm.at[idx_s.at[j2]], rows0, sem0)
            cp1.wait()
            pltpu.sync_copy(rows1, acc.at[idx_d.at[j1]], add=True)
            return carry
        lax.fori_loop(0, CHUNKS // 2, _body, None)
        pltpu.make_async_copy(h_hbm.at[idx_s.at[0]], rows0, sem0).wait()

        plsc.subcore_barrier()
        pltpu.sync_copy(acc.at[pl.ds(s * RPT, RPT)],
                        out_hbm.at[ph, c, pl.ds(s * RPT, RPT)])


def _edge_agg(h, src_r, dst_r):
    mesh = plsc.VectorSubcoreMesh(core_axis_name="c", subcore_axis_name="s")
    fn = pl.kernel(
        _agg_body,
        out_type=jax.ShapeDtypeStruct((2, NC, NPAD, FH), jnp.float32),
        mesh=mesh,
        scratch_types=[
            pltpu.VMEM((CHUNKS, CSZ), jnp.int32),
            pltpu.VMEM((CHUNKS, CSZ), jnp.int32),
            pltpu.VMEM((CSZ, FH), jnp.float32),
            pltpu.VMEM((CSZ, FH), jnp.float32),
            pltpu.VMEM((ZROWS, FH), jnp.float32),
            pltpu.VMEM_SHARED((NPAD, FH), jnp.float32),
            pltpu.SemaphoreType.DMA,
            pltpu.SemaphoreType.DMA,
        ],
        compiler_params=pltpu.CompilerParams(use_tc_tiling_on_sc=False),
        name="gin_edge_agg",
    )
    return fn(h[:, :FH], h[:, FH:], src_r, dst_r)


def _bn(a, g, b):
    mu = jnp.mean(a, axis=0, keepdims=True)
    d = a - mu
    var = jnp.mean(d * d, axis=0, keepdims=True)
    return g * (d * lax.rsqrt(var + EPS)) + b


def _mlp_core(h_ref, p_ref, w1, b1, g1, be1, w2, b2, g2, be2):
    h = h_ref[...]
    agg = jnp.concatenate(
        [p_ref[0, 0, :N, :] + p_ref[0, 1, :N, :],
         p_ref[1, 0, :N, :] + p_ref[1, 1, :N, :]], axis=1)
    m = h + agg
    a = jnp.dot(m, w1[...], preferred_element_type=jnp.float32) + b1[...]
    a = jnp.maximum(_bn(a, g1[...], be1[...]), 0.0)
    a = jnp.dot(a, w2[...], preferred_element_type=jnp.float32) + b2[...]
    a = jnp.maximum(_bn(a, g2[...], be2[...]), 0.0)
    return h, a


def _mlp_first_body(h_ref, p_ref, w1, b1, g1, be1, w2, b2, g2, be2,
                    wfc0, wfc1, hout_ref, zout_ref):
    h, a = _mlp_core(h_ref, p_ref, w1, b1, g1, be1, w2, b2, g2, be2)
    hout_ref[...] = a
    zout_ref[...] = (jnp.dot(h, wfc0[...], preferred_element_type=jnp.float32)
                     + jnp.dot(a, wfc1[...], preferred_element_type=jnp.float32))


def _mlp_mid_body(h_ref, p_ref, w1, b1, g1, be1, w2, b2, g2, be2,
                  wfc, zin_ref, hout_ref, zout_ref):
    _, a = _mlp_core(h_ref, p_ref, w1, b1, g1, be1, w2, b2, g2, be2)
    hout_ref[...] = a
    zout_ref[...] = zin_ref[...] + jnp.dot(
        a, wfc[...], preferred_element_type=jnp.float32)


def _mlp_last_body(h_ref, p_ref, w1, b1, g1, be1, w2, b2, g2, be2,
                   wfc, zin_ref, batch_ref, bias_ref, out_ref):
    _, a = _mlp_core(h_ref, p_ref, w1, b1, g1, be1, w2, b2, g2, be2)
    z = zin_ref[...] + jnp.dot(a, wfc[...], preferred_element_type=jnp.float32)
    row = lax.broadcasted_iota(jnp.int32, (G, N), 0)
    sel = (row == jnp.broadcast_to(batch_ref[...], (G, N))).astype(jnp.float32)
    bias = jnp.sum(bias_ref[...], axis=0, keepdims=True)
    out_ref[...] = jnp.dot(sel, z, preferred_element_type=jnp.float32) + bias


def kernel(x, edge_index, batch, params):
    src_r = edge_index[0].reshape(NW, CHUNKS, CSZ)
    dst_r = edge_index[1].reshape(NW, CHUNKS, CSZ)
    batch2 = batch.reshape(1, N)
    fcs = params['fcs']
    bias_stack = jnp.stack([fcs[i]['b'] for i in range(L + 1)])

    def cp(i):
        p = params['conv%d' % i]
        return (p['W1'], p['b1'].reshape(1, H), p['g1'].reshape(1, H),
                p['be1'].reshape(1, H), p['W2'], p['b2'].reshape(1, H),
                p['g'].reshape(1, H), p['be'].reshape(1, H))

    hz_shape = [jax.ShapeDtypeStruct((N, H), jnp.float32),
                jax.ShapeDtypeStruct((N, OUT), jnp.float32)]

    p = _edge_agg(x, src_r, dst_r)
    h, z = pl.pallas_call(_mlp_first_body, out_shape=hz_shape)(
        x, p, *cp(0), fcs[0]['W'], fcs[1]['W'])

    for i in (1, 2):
        p = _edge_agg(h, src_r, dst_r)
        h, z = pl.pallas_call(_mlp_mid_body, out_shape=hz_shape)(
            h, p, *cp(i), fcs[i + 1]['W'], z)

    p = _edge_agg(h, src_r, dst_r)
    out = pl.pallas_call(
        _mlp_last_body,
        out_shape=jax.ShapeDtypeStruct((G, OUT), jnp.float32))(
        h, p, *cp(3), fcs[4]['W'], z, batch2, bias_stack)
    return out


# speedup vs baseline: 11.1402x; 1.4761x over previous
import functools

import jax
import jax.numpy as jnp
from jax import lax
from jax.experimental import pallas as pl
from jax.experimental.pallas import tpu as pltpu
from jax.experimental.pallas import tpu_sc as plsc

N = 10000
E = 320000
D = 128
H = 128
OUT = 64
G = 128
L = 4
EPS = 1e-5

NC = 2
NS = 16
NW = NC * NS
EW = E // NW
CSZ = 100
CHUNKS = EW // CSZ
NPAD = 10240
RPT = NPAD // NS
ZROWS = 80


def _agg_body(h_hbm, src_hbm, dst_hbm, out_hbm,
              idx_s, idx_d, rows0, rows1, acc, sem0, sem1):
    c = lax.axis_index("c")
    s = lax.axis_index("s")
    wid = s * NC + c

    def _zb(k, carry):
        rows0[k // 8, pl.ds((k % 8) * 16, 16)] = jnp.zeros((16,), jnp.float32)
        return carry
    lax.fori_loop(0, ZROWS * 8, _zb, None)

    def _zc(t, carry):
        pltpu.sync_copy(rows0.at[pl.ds(0, ZROWS)],
                        acc.at[pl.ds(s * RPT + t * ZROWS, ZROWS)])
        return carry
    lax.fori_loop(0, RPT // ZROWS, _zc, None)
    plsc.subcore_barrier()

    pltpu.sync_copy(src_hbm.at[wid], idx_s)
    pltpu.sync_copy(dst_hbm.at[wid], idx_d)

    pltpu.async_copy(h_hbm.at[idx_s.at[0]], rows0, sem0)

    def _body(jj, carry):
        j0 = 2 * jj
        j1 = j0 + 1
        j2 = lax.rem(j0 + 2, CHUNKS)
        cp1 = pltpu.async_copy(h_hbm.at[idx_s.at[j1]], rows1, sem1)
        pltpu.make_async_copy(h_hbm.at[idx_s.at[j0]], rows0, sem0).wait()
        pltpu.sync_copy(rows0, acc.at[idx_d.at[j0]], add=True)
        pltpu.async_copy(h_hbm.at[idx_s.at[j2]], rows0, sem0)
        cp1.wait()
        pltpu.sync_copy(rows1, acc.at[idx_d.at[j1]], add=True)
        return carry
    lax.fori_loop(0, CHUNKS // 2, _body, None)
    pltpu.make_async_copy(h_hbm.at[idx_s.at[0]], rows0, sem0).wait()

    plsc.subcore_barrier()
    pltpu.sync_copy(acc.at[pl.ds(s * RPT, RPT)],
                    out_hbm.at[c, pl.ds(s * RPT, RPT)])


def _edge_agg(h, src_r, dst_r):
    mesh = plsc.VectorSubcoreMesh(core_axis_name="c", subcore_axis_name="s")
    fn = pl.kernel(
        _agg_body,
        out_type=jax.ShapeDtypeStruct((NC, NPAD, H), jnp.float32),
        mesh=mesh,
        scratch_types=[
            pltpu.VMEM((CHUNKS, CSZ), jnp.int32),
            pltpu.VMEM((CHUNKS, CSZ), jnp.int32),
            pltpu.VMEM((CSZ, H), jnp.float32),
            pltpu.VMEM((CSZ, H), jnp.float32),
            pltpu.VMEM_SHARED((NPAD, H), jnp.float32),
            pltpu.SemaphoreType.DMA,
            pltpu.SemaphoreType.DMA,
        ],
        compiler_params=pltpu.CompilerParams(use_tc_tiling_on_sc=False),
        name="gin_edge_agg",
    )
    return fn(h, src_r, dst_r)


def _bn(a, g, b):
    mu = jnp.mean(a, axis=0, keepdims=True)
    d = a - mu
    var = jnp.mean(d * d, axis=0, keepdims=True)
    return g * (d * lax.rsqrt(var + EPS)) + b


def _mlp_core(h_ref, p_ref, w1, b1, g1, be1, w2, b2, g2, be2):
    h = h_ref[...]
    m = h + p_ref[0, :N, :] + p_ref[1, :N, :]
    a = jnp.dot(m, w1[...], preferred_element_type=jnp.float32) + b1[...]
    a = jnp.maximum(_bn(a, g1[...], be1[...]), 0.0)
    a = jnp.dot(a, w2[...], preferred_element_type=jnp.float32) + b2[...]
    a = jnp.maximum(_bn(a, g2[...], be2[...]), 0.0)
    return h, a


def _mlp_first_body(h_ref, p_ref, w1, b1, g1, be1, w2, b2, g2, be2,
                    wfc0, wfc1, hout_ref, zout_ref):
    h, a = _mlp_core(h_ref, p_ref, w1, b1, g1, be1, w2, b2, g2, be2)
    hout_ref[...] = a
    zout_ref[...] = (jnp.dot(h, wfc0[...], preferred_element_type=jnp.float32)
                     + jnp.dot(a, wfc1[...], preferred_element_type=jnp.float32))


def _mlp_mid_body(h_ref, p_ref, w1, b1, g1, be1, w2, b2, g2, be2,
                  wfc, zin_ref, hout_ref, zout_ref):
    _, a = _mlp_core(h_ref, p_ref, w1, b1, g1, be1, w2, b2, g2, be2)
    hout_ref[...] = a
    zout_ref[...] = zin_ref[...] + jnp.dot(
        a, wfc[...], preferred_element_type=jnp.float32)


def _mlp_last_body(h_ref, p_ref, w1, b1, g1, be1, w2, b2, g2, be2,
                   wfc, zin_ref, batch_ref, bias_ref, out_ref):
    _, a = _mlp_core(h_ref, p_ref, w1, b1, g1, be1, w2, b2, g2, be2)
    z = zin_ref[...] + jnp.dot(a, wfc[...], preferred_element_type=jnp.float32)
    row = lax.broadcasted_iota(jnp.int32, (G, N), 0)
    sel = (row == jnp.broadcast_to(batch_ref[...], (G, N))).astype(jnp.float32)
    bias = jnp.sum(bias_ref[...], axis=0, keepdims=True)
    out_ref[...] = jnp.dot(sel, z, preferred_element_type=jnp.float32) + bias


def kernel(x, edge_index, batch, params):
    src_r = edge_index[0].reshape(NW, CHUNKS, CSZ)
    dst_r = edge_index[1].reshape(NW, CHUNKS, CSZ)
    batch2 = batch.reshape(1, N)
    fcs = params['fcs']
    bias_stack = jnp.stack([fcs[i]['b'] for i in range(L + 1)])

    def cp(i):
        p = params['conv%d' % i]
        return (p['W1'], p['b1'].reshape(1, H), p['g1'].reshape(1, H),
                p['be1'].reshape(1, H), p['W2'], p['b2'].reshape(1, H),
                p['g'].reshape(1, H), p['be'].reshape(1, H))

    hz_shape = [jax.ShapeDtypeStruct((N, H), jnp.float32),
                jax.ShapeDtypeStruct((N, OUT), jnp.float32)]

    p = _edge_agg(x, src_r, dst_r)
    h, z = pl.pallas_call(_mlp_first_body, out_shape=hz_shape)(
        x, p, *cp(0), fcs[0]['W'], fcs[1]['W'])

    for i in (1, 2):
        p = _edge_agg(h, src_r, dst_r)
        h, z = pl.pallas_call(_mlp_mid_body, out_shape=hz_shape)(
            h, p, *cp(i), fcs[i + 1]['W'], z)

    p = _edge_agg(h, src_r, dst_r)
    out = pl.pallas_call(
        _mlp_last_body,
        out_shape=jax.ShapeDtypeStruct((G, OUT), jnp.float32))(
        h, p, *cp(3), fcs[4]['W'], z, batch2, bias_stack)
    return out
